# baseline (speedup 1.0000x reference)
import jax
import jax.numpy as jnp
from jax.experimental import pallas as pl

H, W = 512, 512


def _copy_body(x_ref, o_ref):
    o_ref[...] = x_ref[...]


def kernel(points, colors):
    B, _, N = points.shape
    points = pl.pallas_call(
        _copy_body,
        out_shape=jax.ShapeDtypeStruct(points.shape, points.dtype),
    )(points)
    u = jnp.clip(jnp.floor(points[:, 0] * W).astype(jnp.int32), 0, W - 1)
    v = jnp.clip(jnp.floor(points[:, 1] * H).astype(jnp.int32), 0, H - 1)
    z = points[:, 2]
    pix = v * W + u

    def per_batch(pix_b, z_b, col_b):
        d = jnp.full((H * W,), jnp.inf, dtype=z_b.dtype).at[pix_b].min(z_b)
        ids = jnp.arange(N, dtype=jnp.int32)
        match = z_b == d[pix_b]
        ids_m = jnp.where(match, ids, jnp.int32(N))
        idx = jnp.full((H * W,), N, dtype=jnp.int32).at[pix_b].min(ids_m)
        valid = idx < N
        idx_safe = jnp.where(valid, idx, 0)
        depth_map = jnp.where(valid, jnp.take(z_b, idx_safe), 0.0)
        img = jnp.where(valid[None, :], jnp.take(col_b, idx_safe, axis=1), 0.0)
        index = jnp.where(valid, idx, -1)
        return depth_map.reshape(H, W), img.reshape(col_b.shape[0], H, W), index.reshape(H, W)

    depth_map, img, index = jax.vmap(per_batch)(pix, z, colors)
    return depth_map, img, index


# speedup vs baseline: 26.8794x; 26.8794x over previous
import jax
import jax.numpy as jnp
from jax import lax
from jax.experimental import pallas as pl
from jax.experimental.pallas import tpu as pltpu
from jax.experimental.pallas import tpu_sc as plsc

H = 512
W = 512
HW = H * W
NPT = 131072
NB = 8
NSUB = 8
SLAB = HW // NSUB
SLAB_SHIFT = 15
CH = 4096
NCHUNK = NPT // CH
L = 16


def _scatter_min(buf_ref, loc, val, want0):
    def cond(w):
        return jnp.max(w) > 0

    def body(w):
        m = w > 0
        plsc.store_scatter(buf_ref, [loc], val, mask=m)
        cur = plsc.load_gather(buf_ref, [loc])
        return (m & (val < cur)).astype(jnp.int32)

    lax.while_loop(cond, body, want0)


def _body(pts_ref, col_ref, depth_hbm, img_hbm, index_hbm,
          depth_ref, idbuf_ref, imgc_ref, x_ref, y_ref, z_ref, c_ref, sem):
    cax = lax.axis_index("c")
    s = lax.axis_index("s")
    gid = cax * 2 + s // NSUB
    o = s % NSUB
    iota = lax.broadcasted_iota(jnp.int32, (L,), 0)

    for t in range(2):
        b = gid * 2 + t
        pbase = b * 4 * NPT

        def init_body(i, _):
            depth_ref[pl.ds(i * L, L)] = jnp.full((L,), jnp.inf, jnp.float32)
            idbuf_ref[pl.ds(i * L, L)] = jnp.full((L,), NPT, jnp.int32)
            return 0

        lax.fori_loop(0, SLAB // L, init_body, 0)

        def make_scan(mode, ch=0):
            def chunk_body(ci, _):
                off = ci * CH
                hs = [pltpu.async_copy(
                    pts_ref.at[pl.ds(pbase + r * NPT + off, CH)], buf, sem)
                    for r, buf in ((0, x_ref), (1, y_ref), (2, z_ref))]
                if mode == 3:
                    hs.append(pltpu.async_copy(
                        col_ref.at[pl.ds((b * 3 + ch) * NPT + off, CH)],
                        c_ref, sem))
                for h in hs:
                    h.wait()

                def vec_body(j, _):
                    dsl = pl.ds(j * L, L)
                    xv = x_ref[dsl]
                    yv = y_ref[dsl]
                    zv = z_ref[dsl]
                    u = jnp.minimum(jnp.maximum(
                        (xv * W).astype(jnp.int32), 0), W - 1)
                    v = jnp.minimum(jnp.maximum(
                        (yv * H).astype(jnp.int32), 0), H - 1)
                    pix = v * W + u
                    mine = lax.shift_right_logical(pix, SLAB_SHIFT) == o
                    loc = pix & (SLAB - 1)
                    if mode == 1:
                        cur = plsc.load_gather(depth_ref, [loc])
                        w0 = (mine & (zv < cur)).astype(jnp.int32)
                        _scatter_min(depth_ref, loc, zv, w0)
                    elif mode == 2:
                        curz = plsc.load_gather(depth_ref, [loc])
                        match = mine & (zv == curz)
                        idv = (off + j * L) + iota
                        curi = plsc.load_gather(idbuf_ref, [loc])
                        w0 = (match & (idv < curi)).astype(jnp.int32)
                        _scatter_min(idbuf_ref, loc, idv, w0)
                    else:
                        curz = plsc.load_gather(depth_ref, [loc])
                        curi = plsc.load_gather(idbuf_ref, [loc])
                        idv = (off + j * L) + iota
                        win = mine & (zv == curz) & (idv == curi)
                        cv = c_ref[dsl]
                        plsc.store_scatter(imgc_ref, [loc], cv, mask=win)
                    return 0

                lax.fori_loop(0, CH // L, vec_body, 0)
                return 0

            return chunk_body

        lax.fori_loop(0, NCHUNK, make_scan(1), 0)
        lax.fori_loop(0, NCHUNK, make_scan(2), 0)

        for ch in range(3):
            def zero_body(i, _):
                imgc_ref[pl.ds(i * L, L)] = jnp.zeros((L,), jnp.float32)
                return 0

            lax.fori_loop(0, SLAB // L, zero_body, 0)
            lax.fori_loop(0, NCHUNK, make_scan(3, ch), 0)
            ibase = (b * 3 + ch) * HW + o * SLAB
            pltpu.sync_copy(imgc_ref, img_hbm.at[pl.ds(ibase, SLAB)])

        def fin_body(i, _):
            dsl = pl.ds(i * L, L)
            idv = idbuf_ref[dsl]
            dv = depth_ref[dsl]
            valid = idv < NPT
            depth_ref[dsl] = jnp.where(valid, dv, jnp.float32(0.0))
            idbuf_ref[dsl] = jnp.where(valid, idv, jnp.int32(-1))
            return 0

        lax.fori_loop(0, SLAB // L, fin_body, 0)
        gbase = b * HW + o * SLAB
        pltpu.sync_copy(depth_ref, depth_hbm.at[pl.ds(gbase, SLAB)])
        pltpu.sync_copy(idbuf_ref, index_hbm.at[pl.ds(gbase, SLAB)])


_proj = pl.kernel(
    _body,
    out_type=(
        jax.ShapeDtypeStruct((NB * HW,), jnp.float32),
        jax.ShapeDtypeStruct((NB * 3 * HW,), jnp.float32),
        jax.ShapeDtypeStruct((NB * HW,), jnp.int32),
    ),
    mesh=plsc.VectorSubcoreMesh(core_axis_name="c", subcore_axis_name="s"),
    scratch_types=[
        pltpu.VMEM((SLAB,), jnp.float32),
        pltpu.VMEM((SLAB,), jnp.int32),
        pltpu.VMEM((SLAB,), jnp.float32),
        pltpu.VMEM((CH,), jnp.float32),
        pltpu.VMEM((CH,), jnp.float32),
        pltpu.VMEM((CH,), jnp.float32),
        pltpu.VMEM((CH,), jnp.float32),
        pltpu.SemaphoreType.DMA,
    ],
    compiler_params=pltpu.CompilerParams(needs_layout_passes=False),
)


def kernel(points, colors):
    B, _, N = points.shape
    depth, img, index = _proj(points.reshape(-1), colors.reshape(-1))
    return (depth.reshape(B, H, W),
            img.reshape(B, 3, H, W),
            index.reshape(B, H, W))
